# initial kernel scaffold (unmeasured)
import jax
import jax.numpy as jnp
from jax import lax
from jax.experimental import pallas as pl
from jax.experimental.pallas import tpu as pltpu

N_DEV = 8


def kernel(x, w_mat):
    m, k = x.shape
    k2, n = w_mat.shape
    ch = n // N_DEV

    def body(x_ref, w_ref, out_ref, send_buf, recv_buf,
             send_sem, recv_sem, copy_sem, credit_sem):
        my = lax.axis_index("i")
        left = lax.rem(my - 1 + N_DEV, N_DEV)
        right = lax.rem(my + 1, N_DEV)

        barrier_sem = pltpu.get_barrier_semaphore()
        for nbr in (left, right):
            pl.semaphore_signal(barrier_sem, inc=1, device_id=(nbr,),
                                device_id_type=pl.DeviceIdType.MESH)
        pl.semaphore_wait(barrier_sem, 2)

        def local_chunk(c):
            return jnp.dot(x_ref[...], w_ref[:, pl.ds(c * ch, ch)],
                           preferred_element_type=jnp.float32)

        send_buf[...] = local_chunk(my)

        for s in range(2 * (N_DEV - 1)):
            if s > 0:
                pl.semaphore_wait(credit_sem, 1)
            rdma = pltpu.make_async_remote_copy(
                src_ref=send_buf, dst_ref=recv_buf,
                send_sem=send_sem, recv_sem=recv_sem,
                device_id=(right,), device_id_type=pl.DeviceIdType.MESH)
            rdma.start()
            rdma.wait()

            if s < N_DEV - 1:
                c = lax.rem(my - s - 1 + N_DEV, N_DEV)
                send_buf[...] = recv_buf[...] + local_chunk(c)
                if s == N_DEV - 2:
                    cp = pltpu.make_async_copy(
                        send_buf, out_ref.at[:, pl.ds(c * ch, ch)], copy_sem)
                    cp.start()
                    cp.wait()
            else:
                t = s - (N_DEV - 1)
                c = lax.rem(my - t + N_DEV, N_DEV)
                cp = pltpu.make_async_copy(
                    recv_buf, out_ref.at[:, pl.ds(c * ch, ch)], copy_sem)
                cp.start()
                cp.wait()
                if s < 2 * (N_DEV - 1) - 1:
                    send_buf[...] = recv_buf[...]

            if s < 2 * (N_DEV - 1) - 1:
                pl.semaphore_signal(credit_sem, inc=1, device_id=(left,),
                                    device_id_type=pl.DeviceIdType.MESH)

    return pl.pallas_call(
        body,
        out_shape=jax.ShapeDtypeStruct((m, n), jnp.float32),
        in_specs=[pl.BlockSpec(memory_space=pltpu.VMEM),
                  pl.BlockSpec(memory_space=pltpu.VMEM)],
        out_specs=pl.BlockSpec(memory_space=pltpu.ANY),
        scratch_shapes=[
            pltpu.VMEM((m, ch), jnp.float32),
            pltpu.VMEM((m, ch), jnp.float32),
            pltpu.SemaphoreType.DMA,
            pltpu.SemaphoreType.DMA,
            pltpu.SemaphoreType.DMA,
            pltpu.SemaphoreType.REGULAR,
        ],
        compiler_params=pltpu.CompilerParams(collective_id=0),
    )(x, w_mat)


# baseline (device time: 2754466 ns/iter reference)
import jax
import jax.numpy as jnp
from jax import lax
from jax.experimental import pallas as pl
from jax.experimental.pallas import tpu as pltpu

N_DEV = 8


def kernel(x, w_mat):
    m, k = x.shape
    k2, n = w_mat.shape
    ch = n // N_DEV

    def body(x_ref, w_ref, out_ref, send_buf, recv_buf,
             send_sem, recv_sem, copy_sem, credit_sem):
        my = lax.axis_index("i")
        left = lax.rem(my - 1 + N_DEV, N_DEV)
        right = lax.rem(my + 1, N_DEV)

        barrier_sem = pltpu.get_barrier_semaphore()
        for nbr in (left, right):
            pl.semaphore_signal(barrier_sem, inc=1, device_id=(nbr,),
                                device_id_type=pl.DeviceIdType.MESH)
        pl.semaphore_wait(barrier_sem, 2)

        def local_chunk(c):
            return jnp.dot(x_ref[...], w_ref[:, pl.ds(c * ch, ch)],
                           preferred_element_type=jnp.float32)

        send_buf[...] = local_chunk(my)

        for s in range(2 * (N_DEV - 1)):
            if s > 0:
                pl.semaphore_wait(credit_sem, 1)
            rdma = pltpu.make_async_remote_copy(
                src_ref=send_buf, dst_ref=recv_buf,
                send_sem=send_sem, recv_sem=recv_sem,
                device_id=(right,), device_id_type=pl.DeviceIdType.MESH)
            rdma.start()
            rdma.wait()

            if s < N_DEV - 1:
                c = lax.rem(my - s - 1 + N_DEV, N_DEV)
                send_buf[...] = recv_buf[...] + local_chunk(c)
                if s == N_DEV - 2:
                    cp = pltpu.make_async_copy(
                        send_buf, out_ref.at[:, pl.ds(c * ch, ch)], copy_sem)
                    cp.start()
                    cp.wait()
            else:
                t = s - (N_DEV - 1)
                c = lax.rem(my - t + N_DEV, N_DEV)
                cp = pltpu.make_async_copy(
                    recv_buf, out_ref.at[:, pl.ds(c * ch, ch)], copy_sem)
                cp.start()
                cp.wait()
                if s < 2 * (N_DEV - 1) - 1:
                    send_buf[...] = recv_buf[...]

            if s < 2 * (N_DEV - 1) - 1:
                pl.semaphore_signal(credit_sem, inc=1, device_id=(left,),
                                    device_id_type=pl.DeviceIdType.MESH)

    return pl.pallas_call(
        body,
        out_shape=jax.ShapeDtypeStruct((m, n), jnp.float32),
        in_specs=[pl.BlockSpec(memory_space=pltpu.VMEM),
                  pl.BlockSpec(memory_space=pltpu.VMEM)],
        out_specs=pl.BlockSpec(memory_space=pl.ANY),
        scratch_shapes=[
            pltpu.VMEM((m, ch), jnp.float32),
            pltpu.VMEM((m, ch), jnp.float32),
            pltpu.SemaphoreType.DMA,
            pltpu.SemaphoreType.DMA,
            pltpu.SemaphoreType.DMA,
            pltpu.SemaphoreType.REGULAR,
        ],
        compiler_params=pltpu.CompilerParams(
            collective_id=0, vmem_limit_bytes=100 * 1024 * 1024),
    )(x, w_mat)


# device time: 1429713 ns/iter; 1.9266x vs baseline; 1.9266x over previous
import jax
import jax.numpy as jnp
from jax import lax
from jax.experimental import pallas as pl
from jax.experimental.pallas import tpu as pltpu

N_DEV = 8
NH = N_DEV - 1


def kernel(x, w_mat):
    m, k = x.shape
    _, n = w_mat.shape
    ch = n // N_DEV
    h = ch // 2

    def body(x_ref, w_ref, out_ref, acc, rcv,
             rs_send_sems, rs_recv_sems, ag_send_sems, ag_recv_sems,
             copy_sems, credit_f, credit_b):
        my = lax.axis_index("i")
        left = lax.rem(my - 1 + N_DEV, N_DEV)
        right = lax.rem(my + 1, N_DEV)

        barrier_sem = pltpu.get_barrier_semaphore()
        for nbr in (left, right):
            pl.semaphore_signal(barrier_sem, inc=1, device_id=(nbr,),
                                device_id_type=pl.DeviceIdType.MESH)
        pl.semaphore_wait(barrier_sem, 2)

        tgt = (right, left)
        ups = (left, right)
        credit = (credit_f, credit_b)

        def col(c, d):
            return c * ch + d * h

        def dot(c, d):
            return jnp.dot(x_ref[...], w_ref[:, pl.ds(col(c, d), h)],
                           preferred_element_type=jnp.float32)

        def cons_chunk(s, d):
            if d == 0:
                return lax.rem(my - s - 1 + N_DEV, N_DEV)
            return lax.rem(my + s + 1, N_DEV)

        def recv_chunk(t, d):
            if d == 0:
                return lax.rem(my - t + N_DEV, N_DEV)
            return lax.rem(my + t, N_DEV)

        for d in (0, 1):
            acc[d, :, :] = dot(my, d)

        rs = [[None] * NH for _ in range(2)]
        for s in range(NH):
            for d in (0, 1):
                if s >= 1:
                    pl.semaphore_wait(credit[d], 1)
                r = pltpu.make_async_remote_copy(
                    src_ref=acc.at[d],
                    dst_ref=rcv.at[d],
                    send_sem=rs_send_sems.at[d, s],
                    recv_sem=rs_recv_sems.at[d, s],
                    device_id=(tgt[d],), device_id_type=pl.DeviceIdType.MESH)
                r.start()
                rs[d][s] = r
            for d in (0, 1):
                rs[d][s].wait_send()
                acc[d, :, :] = dot(cons_chunk(s, d), d)
            for d in (0, 1):
                rs[d][s].wait_recv()
                acc[d, :, :] = acc[d, :, :] + rcv[d, :, :]
                if s <= NH - 2:
                    pl.semaphore_signal(credit[d], inc=1, device_id=(ups[d],),
                                        device_id_type=pl.DeviceIdType.MESH)

        ag = [[None] * NH for _ in range(2)]
        cps = [None, None]
        for d in (0, 1):
            g = cons_chunk(NH - 1, d)
            cp = pltpu.make_async_copy(
                acc.at[d], out_ref.at[:, pl.ds(col(g, d), h)],
                copy_sems.at[d])
            cp.start()
            cps[d] = cp
            r = pltpu.make_async_remote_copy(
                src_ref=acc.at[d],
                dst_ref=out_ref.at[:, pl.ds(col(g, d), h)],
                send_sem=ag_send_sems.at[d, 0],
                recv_sem=ag_recv_sems.at[d, 0],
                device_id=(tgt[d],), device_id_type=pl.DeviceIdType.MESH)
            r.start()
            ag[d][0] = r
        for t in range(1, NH):
            for d in (0, 1):
                ag[d][t - 1].wait_recv()
                c = recv_chunk(t - 1, d)
                r = pltpu.make_async_remote_copy(
                    src_ref=out_ref.at[:, pl.ds(col(c, d), h)],
                    dst_ref=out_ref.at[:, pl.ds(col(c, d), h)],
                    send_sem=ag_send_sems.at[d, t],
                    recv_sem=ag_recv_sems.at[d, t],
                    device_id=(tgt[d],), device_id_type=pl.DeviceIdType.MESH)
                r.start()
                ag[d][t] = r

        for d in (0, 1):
            ag[d][NH - 1].wait_recv()
            for t in range(NH):
                ag[d][t].wait_send()
            cps[d].wait()

    return pl.pallas_call(
        body,
        out_shape=jax.ShapeDtypeStruct((m, n), jnp.float32),
        in_specs=[pl.BlockSpec(memory_space=pltpu.VMEM),
                  pl.BlockSpec(memory_space=pltpu.VMEM)],
        out_specs=pl.BlockSpec(memory_space=pl.ANY),
        scratch_shapes=[
            pltpu.VMEM((2, m, h), jnp.float32),
            pltpu.VMEM((2, m, h), jnp.float32),
            pltpu.SemaphoreType.DMA((2, NH)),
            pltpu.SemaphoreType.DMA((2, NH)),
            pltpu.SemaphoreType.DMA((2, NH)),
            pltpu.SemaphoreType.DMA((2, NH)),
            pltpu.SemaphoreType.DMA((2,)),
            pltpu.SemaphoreType.REGULAR,
            pltpu.SemaphoreType.REGULAR,
        ],
        compiler_params=pltpu.CompilerParams(
            collective_id=0, vmem_limit_bytes=100 * 1024 * 1024),
    )(x, w_mat)


# device time: 1363812 ns/iter; 2.0197x vs baseline; 1.0483x over previous
import jax
import jax.numpy as jnp
from jax import lax
from jax.experimental import pallas as pl
from jax.experimental.pallas import tpu as pltpu

N_DEV = 8
NH = N_DEV - 1
NG = 2


def kernel(x, w_mat):
    m, k = x.shape
    _, n = w_mat.shape
    ch = n // N_DEV
    h = ch // 2
    mg = m // NG

    def body(x_ref, w_ref, out_ref, acc, rcv,
             rs_send_sems, rs_recv_sems, ag_send_sems, ag_recv_sems,
             copy_sems, credit_sems):
        my = lax.axis_index("i")
        left = lax.rem(my - 1 + N_DEV, N_DEV)
        right = lax.rem(my + 1, N_DEV)

        barrier_sem = pltpu.get_barrier_semaphore()
        for nbr in (left, right):
            pl.semaphore_signal(barrier_sem, inc=1, device_id=(nbr,),
                                device_id_type=pl.DeviceIdType.MESH)
        pl.semaphore_wait(barrier_sem, 2)

        tgt = (right, left)
        ups = (left, right)

        def col(c, d):
            return c * ch + d * h

        def cons_chunk(s, d):
            if d == 0:
                return lax.rem(my - s - 1 + N_DEV, N_DEV)
            return lax.rem(my + s + 1, N_DEV)

        def recv_chunk(t, d):
            if d == 0:
                return lax.rem(my - t + N_DEV, N_DEV)
            return lax.rem(my + t, N_DEV)

        def dot_g(c, d, g):
            return jnp.dot(x_ref[pl.ds(g * mg, mg), :],
                           w_ref[:, pl.ds(col(c, d), h)],
                           preferred_element_type=jnp.float32)

        def mk_rs(d, s, g):
            return pltpu.make_async_remote_copy(
                src_ref=acc.at[d, pl.ds(g * mg, mg)],
                dst_ref=rcv.at[d, pl.ds(g * mg, mg)],
                send_sem=rs_send_sems.at[d, s, g],
                recv_sem=rs_recv_sems.at[d, s, g],
                device_id=(tgt[d],), device_id_type=pl.DeviceIdType.MESH)

        def mk_ag(d, t, g, c, from_acc=False):
            src = (acc.at[d, pl.ds(g * mg, mg)] if from_acc else
                   out_ref.at[pl.ds(g * mg, mg), pl.ds(col(c, d), h)])
            return pltpu.make_async_remote_copy(
                src_ref=src,
                dst_ref=out_ref.at[pl.ds(g * mg, mg), pl.ds(col(c, d), h)],
                send_sem=ag_send_sems.at[d, t, g],
                recv_sem=ag_recv_sems.at[d, t, g],
                device_id=(tgt[d],), device_id_type=pl.DeviceIdType.MESH)

        rs = [[[None] * NG for _ in range(NH)] for _ in range(2)]
        ag = [[[None] * NG for _ in range(NH)] for _ in range(2)]
        cps = [[None] * NG, [None] * NG]

        for g in range(NG):
            for d in (0, 1):
                acc[d, pl.ds(g * mg, mg), :] = dot_g(my, d, g)
            for d in (0, 1):
                r = mk_rs(d, 0, g)
                r.start()
                rs[d][0][g] = r

        for s in range(NH):
            for g in range(NG):
                for d in (0, 1):
                    rs[d][s][g].wait_send()
                    rs[d][s][g].wait_recv()
                    acc[d, pl.ds(g * mg, mg), :] = (
                        rcv[d, pl.ds(g * mg, mg), :]
                        + dot_g(cons_chunk(s, d), d, g))
                    if s < NH - 1:
                        pl.semaphore_signal(
                            credit_sems.at[d, g], inc=1, device_id=(ups[d],),
                            device_id_type=pl.DeviceIdType.MESH)
                        pl.semaphore_wait(credit_sems.at[d, g], 1)
                        r = mk_rs(d, s + 1, g)
                        r.start()
                        rs[d][s + 1][g] = r
                    else:
                        gc = cons_chunk(NH - 1, d)
                        cp = pltpu.make_async_copy(
                            acc.at[d, pl.ds(g * mg, mg)],
                            out_ref.at[pl.ds(g * mg, mg),
                                       pl.ds(col(gc, d), h)],
                            copy_sems.at[d, g])
                        cp.start()
                        cps[d][g] = cp
                        r = mk_ag(d, 0, g, gc, from_acc=True)
                        r.start()
                        ag[d][0][g] = r

        for t in range(1, NH):
            for g in range(NG):
                for d in (0, 1):
                    ag[d][t - 1][g].wait_recv()
                    r = mk_ag(d, t, g, recv_chunk(t - 1, d))
                    r.start()
                    ag[d][t][g] = r

        for d in (0, 1):
            for g in range(NG):
                ag[d][NH - 1][g].wait_recv()
                for t in range(NH):
                    ag[d][t][g].wait_send()
                cps[d][g].wait()

    return pl.pallas_call(
        body,
        out_shape=jax.ShapeDtypeStruct((m, n), jnp.float32),
        in_specs=[pl.BlockSpec(memory_space=pltpu.VMEM),
                  pl.BlockSpec(memory_space=pltpu.VMEM)],
        out_specs=pl.BlockSpec(memory_space=pl.ANY),
        scratch_shapes=[
            pltpu.VMEM((2, m, h), jnp.float32),
            pltpu.VMEM((2, m, h), jnp.float32),
            pltpu.SemaphoreType.DMA((2, NH, NG)),
            pltpu.SemaphoreType.DMA((2, NH, NG)),
            pltpu.SemaphoreType.DMA((2, NH, NG)),
            pltpu.SemaphoreType.DMA((2, NH, NG)),
            pltpu.SemaphoreType.DMA((2, NG)),
            pltpu.SemaphoreType.REGULAR((2, NG)),
        ],
        compiler_params=pltpu.CompilerParams(
            collective_id=0, vmem_limit_bytes=100 * 1024 * 1024),
    )(x, w_mat)


# device time: 1362241 ns/iter; 2.0220x vs baseline; 1.0012x over previous
import jax
import jax.numpy as jnp
from jax import lax
from jax.experimental import pallas as pl
from jax.experimental.pallas import tpu as pltpu

N_DEV = 8
NH = N_DEV - 1
NG = 4


def kernel(x, w_mat):
    m, k = x.shape
    _, n = w_mat.shape
    ch = n // N_DEV
    h = ch // 2
    mg = m // NG

    def body(x_ref, w_ref, out_ref, acc, rcv,
             rs_send_sems, rs_recv_sems, ag_send_sems, ag_recv_sems,
             copy_sems, credit_sems):
        my = lax.axis_index("i")
        left = lax.rem(my - 1 + N_DEV, N_DEV)
        right = lax.rem(my + 1, N_DEV)

        barrier_sem = pltpu.get_barrier_semaphore()
        for nbr in (left, right):
            pl.semaphore_signal(barrier_sem, inc=1, device_id=(nbr,),
                                device_id_type=pl.DeviceIdType.MESH)
        pl.semaphore_wait(barrier_sem, 2)

        tgt = (right, left)
        ups = (left, right)

        def col(c, d):
            return c * ch + d * h

        def cons_chunk(s, d):
            if d == 0:
                return lax.rem(my - s - 1 + N_DEV, N_DEV)
            return lax.rem(my + s + 1, N_DEV)

        def recv_chunk(t, d):
            if d == 0:
                return lax.rem(my - t + N_DEV, N_DEV)
            return lax.rem(my + t, N_DEV)

        def dot_g(c, d, g):
            return jnp.dot(x_ref[pl.ds(g * mg, mg), :],
                           w_ref[:, pl.ds(col(c, d), h)],
                           preferred_element_type=jnp.float32)

        def mk_rs(d, s, g):
            return pltpu.make_async_remote_copy(
                src_ref=acc.at[d, pl.ds(g * mg, mg)],
                dst_ref=rcv.at[d, pl.ds(g * mg, mg)],
                send_sem=rs_send_sems.at[d, s, g],
                recv_sem=rs_recv_sems.at[d, s, g],
                device_id=(tgt[d],), device_id_type=pl.DeviceIdType.MESH)

        def mk_ag(d, t, g, c, from_acc=False):
            src = (acc.at[d, pl.ds(g * mg, mg)] if from_acc else
                   out_ref.at[pl.ds(g * mg, mg), pl.ds(col(c, d), h)])
            return pltpu.make_async_remote_copy(
                src_ref=src,
                dst_ref=out_ref.at[pl.ds(g * mg, mg), pl.ds(col(c, d), h)],
                send_sem=ag_send_sems.at[d, t, g],
                recv_sem=ag_recv_sems.at[d, t, g],
                device_id=(tgt[d],), device_id_type=pl.DeviceIdType.MESH)

        rs = [[[None] * NG for _ in range(NH)] for _ in range(2)]
        ag = [[[None] * NG for _ in range(NH)] for _ in range(2)]
        cps = [[None] * NG, [None] * NG]

        for g in range(NG):
            for d in (0, 1):
                acc[d, pl.ds(g * mg, mg), :] = dot_g(my, d, g)
            for d in (0, 1):
                r = mk_rs(d, 0, g)
                r.start()
                rs[d][0][g] = r

        for s in range(NH):
            for g in range(NG):
                for d in (0, 1):
                    rs[d][s][g].wait_send()
                    rs[d][s][g].wait_recv()
                    acc[d, pl.ds(g * mg, mg), :] = (
                        rcv[d, pl.ds(g * mg, mg), :]
                        + dot_g(cons_chunk(s, d), d, g))
                    if s < NH - 1:
                        pl.semaphore_signal(
                            credit_sems.at[d, g], inc=1, device_id=(ups[d],),
                            device_id_type=pl.DeviceIdType.MESH)
                        pl.semaphore_wait(credit_sems.at[d, g], 1)
                        r = mk_rs(d, s + 1, g)
                        r.start()
                        rs[d][s + 1][g] = r
                    else:
                        gc = cons_chunk(NH - 1, d)
                        cp = pltpu.make_async_copy(
                            acc.at[d, pl.ds(g * mg, mg)],
                            out_ref.at[pl.ds(g * mg, mg),
                                       pl.ds(col(gc, d), h)],
                            copy_sems.at[d, g])
                        cp.start()
                        cps[d][g] = cp
                        r = mk_ag(d, 0, g, gc, from_acc=True)
                        r.start()
                        ag[d][0][g] = r

        for t in range(1, NH):
            for g in range(NG):
                for d in (0, 1):
                    ag[d][t - 1][g].wait_recv()
                    r = mk_ag(d, t, g, recv_chunk(t - 1, d))
                    r.start()
                    ag[d][t][g] = r

        for d in (0, 1):
            for g in range(NG):
                ag[d][NH - 1][g].wait_recv()
                for t in range(NH):
                    ag[d][t][g].wait_send()
                cps[d][g].wait()

    return pl.pallas_call(
        body,
        out_shape=jax.ShapeDtypeStruct((m, n), jnp.float32),
        in_specs=[pl.BlockSpec(memory_space=pltpu.VMEM),
                  pl.BlockSpec(memory_space=pltpu.VMEM)],
        out_specs=pl.BlockSpec(memory_space=pl.ANY),
        scratch_shapes=[
            pltpu.VMEM((2, m, h), jnp.float32),
            pltpu.VMEM((2, m, h), jnp.float32),
            pltpu.SemaphoreType.DMA((2, NH, NG)),
            pltpu.SemaphoreType.DMA((2, NH, NG)),
            pltpu.SemaphoreType.DMA((2, NH, NG)),
            pltpu.SemaphoreType.DMA((2, NH, NG)),
            pltpu.SemaphoreType.DMA((2, NG)),
            pltpu.SemaphoreType.REGULAR((2, NG)),
        ],
        compiler_params=pltpu.CompilerParams(
            collective_id=0, vmem_limit_bytes=100 * 1024 * 1024),
    )(x, w_mat)
